# baseline (speedup 1.0000x reference)
import functools

import jax
import jax.numpy as jnp
from jax import lax
from jax.experimental import pallas as pl
from jax.experimental.pallas import tpu as pltpu
from jax.experimental.pallas import tpu_sc as plsc

_INFO = plsc.get_sparse_core_info()
_NC = _INFO.num_cores
_NS = _INFO.num_subcores
_NW = _NC * _NS


@functools.cache
def _make_gather(B, D):
    b_per_w = B // _NW
    CH = 32
    NBUF = 3
    n_ch = b_per_w // CH
    mesh = plsc.VectorSubcoreMesh(core_axis_name="c", subcore_axis_name="s")

    @functools.partial(
        pl.kernel,
        mesh=mesh,
        out_type=jax.ShapeDtypeStruct((B, D), jnp.float32),
        scratch_types=[
            pltpu.VMEM((NBUF, CH, D), jnp.float32),
            pltpu.VMEM((16,), jnp.int32),
        ]
        + [pltpu.SemaphoreType.DMA] * (2 * NBUF),
    )
    def gather_kernel(sl_hbm, table_hbm, out_hbm, bufs, sl_v, *sems):
        gs, ss = sems[:NBUF], sems[NBUF:]
        wid = lax.axis_index("s") * _NC + lax.axis_index("c")
        base = wid * b_per_w
        pltpu.sync_copy(sl_hbm, sl_v)
        sl = sl_v[...]
        sl_scalar = sl[0]
        lanes = lax.iota(jnp.int32, 16)

        def store(j, b):
            return pltpu.async_copy(
                bufs.at[b], out_hbm.at[pl.ds(base + j * CH, CH)], ss[b]
            )

        def run_ring(gather):
            gh, sh = {}, {}
            for b in range(min(NBUF, n_ch)):
                gh[b] = gather(b, b)
            for j in range(n_ch):
                b = j % NBUF
                gh[b].wait()
                sh[b] = store(j, b)
                nxt = j + NBUF
                if nxt < n_ch:
                    sh[b].wait()
                    gh[b] = gather(nxt, b)
            for k in range(max(0, n_ch - NBUF), n_ch):
                sh[k % NBUF].wait()

        @pl.when(sl_scalar >= B)
        def _linear():
            def gather(j, b):
                return pltpu.async_copy(
                    table_hbm.at[pl.ds(base + j * CH, CH)], bufs.at[b], gs[b]
                )

            run_ring(gather)

        @pl.when(sl_scalar < B)
        def _general():
            def gather(j, b):
                r0 = base + j * CH
                idx0 = (lanes + r0) % sl
                idx1 = (lanes + (r0 + 16)) % sl
                pltpu.async_copy(table_hbm.at[idx0], bufs.at[b, pl.ds(0, 16)], gs[b])
                pltpu.async_copy(table_hbm.at[idx1], bufs.at[b, pl.ds(16, 16)], gs[b])
                return pltpu.make_async_copy(
                    table_hbm.at[pl.ds(0, CH)], bufs.at[b], gs[b]
                )

            run_ring(gather)

    return gather_kernel


def kernel(seq_len, table):
    V, D = table.shape
    sl_vec = jnp.broadcast_to(jnp.asarray(seq_len, jnp.int32), (16,))
    return _make_gather(V, D)(sl_vec, table)


# speedup vs baseline: 1.0100x; 1.0100x over previous
import functools

import jax
import jax.numpy as jnp
from jax import lax
from jax.experimental import pallas as pl
from jax.experimental.pallas import tpu as pltpu
from jax.experimental.pallas import tpu_sc as plsc

_INFO = plsc.get_sparse_core_info()
_NC = _INFO.num_cores
_NS = _INFO.num_subcores
_NW = _NC * _NS


@functools.cache
def _make_gather(B, D):
    b_per_w = B // _NW
    CH = 32
    NBUF = 2
    n_ch = b_per_w // CH
    mesh = plsc.VectorSubcoreMesh(core_axis_name="c", subcore_axis_name="s")

    @functools.partial(
        pl.kernel,
        mesh=mesh,
        out_type=jax.ShapeDtypeStruct((B, D), jnp.float32),
        scratch_types=[
            pltpu.VMEM_SHARED((_NS, NBUF, CH, D), jnp.float32),
            pltpu.VMEM((CH, D), jnp.float32),
            pltpu.VMEM((16,), jnp.int32),
        ]
        + [pltpu.SemaphoreType.DMA] * (2 * NBUF + 1),
    )
    def gather_kernel(sl_hbm, table_hbm, out_hbm, bufs, fbuf, sl_v, *sems):
        gs, ss, fsem = sems[:NBUF], sems[NBUF : 2 * NBUF], sems[2 * NBUF]
        sid = lax.axis_index("s")
        wid = sid * _NC + lax.axis_index("c")
        base = wid * b_per_w

        def lin_gather(j, b):
            return pltpu.async_copy(
                table_hbm.at[pl.ds(base + j * CH, CH)], bufs.at[sid, b], gs[b]
            )

        def store(j, b):
            return pltpu.async_copy(
                bufs.at[sid, b], out_hbm.at[pl.ds(base + j * CH, CH)], ss[b]
            )

        gh = {}
        for b in range(min(NBUF, n_ch)):
            gh[b] = lin_gather(b, b)
        pltpu.sync_copy(sl_hbm, sl_v)
        sl = sl_v[...]
        sl_scalar = sl[0]
        lanes = lax.iota(jnp.int32, 16)

        @pl.when(sl_scalar >= B)
        def _linear():
            sh = {}
            for j in range(n_ch):
                b = j % NBUF
                gh[b].wait()
                sh[b] = store(j, b)
                nxt = j + NBUF
                if nxt < n_ch:
                    sh[b].wait()
                    lin_gather(nxt, b)
            for k in range(max(0, n_ch - NBUF), n_ch):
                sh[k % NBUF].wait()

        @pl.when(sl_scalar < B)
        def _general():
            for b in range(min(NBUF, n_ch)):
                gh[b].wait()
            for j in range(n_ch):
                r0 = base + j * CH
                idx0 = (lanes + r0) % sl
                idx1 = (lanes + (r0 + 16)) % sl
                pltpu.async_copy(table_hbm.at[idx0], fbuf.at[pl.ds(0, 16)], fsem)
                pltpu.async_copy(table_hbm.at[idx1], fbuf.at[pl.ds(16, 16)], fsem)
                pltpu.make_async_copy(
                    table_hbm.at[pl.ds(0, CH)], fbuf, fsem
                ).wait()
                pltpu.sync_copy(fbuf, out_hbm.at[pl.ds(r0, CH)])

    return gather_kernel


def kernel(seq_len, table):
    V, D = table.shape
    sl_vec = jnp.broadcast_to(jnp.asarray(seq_len, jnp.int32), (16,))
    return _make_gather(V, D)(sl_vec, table)


# speedup vs baseline: 1.0743x; 1.0637x over previous
import functools

import jax
import jax.numpy as jnp
from jax import lax
from jax.experimental import pallas as pl
from jax.experimental.pallas import tpu as pltpu
from jax.experimental.pallas import tpu_sc as plsc

_INFO = plsc.get_sparse_core_info()
_NC = _INFO.num_cores
_NS = _INFO.num_subcores
_NW = _NC * _NS


@functools.cache
def _make_gather(B, D):
    b_per_w = B // _NW
    CH = 32
    NBUF = 3
    n_ch = b_per_w // CH
    mesh = plsc.VectorSubcoreMesh(core_axis_name="c", subcore_axis_name="s")

    @functools.partial(
        pl.kernel,
        mesh=mesh,
        out_type=jax.ShapeDtypeStruct((B, D), jnp.float32),
        scratch_types=[
            pltpu.VMEM((b_per_w,), jnp.int32),
            pltpu.VMEM((NBUF, CH, D), jnp.float32),
        ]
        + [pltpu.SemaphoreType.DMA] * (2 * NBUF),
    )
    def gather_kernel(idx_hbm, table_hbm, out_hbm, idx_v, bufs, *sems):
        gs, ss = sems[:NBUF], sems[NBUF:]
        wid = lax.axis_index("s") * _NC + lax.axis_index("c")
        base = wid * b_per_w
        pltpu.sync_copy(idx_hbm.at[pl.ds(base, b_per_w)], idx_v)

        def gather(j, b):
            return pltpu.async_copy(
                table_hbm.at[idx_v.at[pl.ds(j * CH, CH)]], bufs.at[b], gs[b]
            )

        def store(j, b):
            return pltpu.async_copy(
                bufs.at[b], out_hbm.at[pl.ds(base + j * CH, CH)], ss[b]
            )

        gh, sh = {}, {}
        for b in range(min(NBUF, n_ch)):
            gh[b] = gather(b, b)
        for j in range(n_ch):
            b = j % NBUF
            gh[b].wait()
            sh[b] = store(j, b)
            nxt = j + NBUF
            if nxt < n_ch:
                sh[b].wait()
                gh[b] = gather(nxt, b)
        for k in range(max(0, n_ch - NBUF), n_ch):
            sh[k % NBUF].wait()

    return gather_kernel


def kernel(seq_len, table):
    V, D = table.shape
    idx = jnp.arange(V, dtype=jnp.int32) % jnp.asarray(seq_len, jnp.int32)
    return _make_gather(V, D)(idx, table)


# speedup vs baseline: 1.0760x; 1.0016x over previous
import functools

import jax
import jax.numpy as jnp
from jax import lax
from jax.experimental import pallas as pl
from jax.experimental.pallas import tpu as pltpu
from jax.experimental.pallas import tpu_sc as plsc

_INFO = plsc.get_sparse_core_info()
_NC = _INFO.num_cores
_NS = _INFO.num_subcores
_NW = _NC * _NS


@functools.cache
def _make_gather(B, D):
    b_per_w = B // _NW
    CH = 16
    NBUF = 7
    n_ch = b_per_w // CH
    mesh = plsc.VectorSubcoreMesh(core_axis_name="c", subcore_axis_name="s")

    @functools.partial(
        pl.kernel,
        mesh=mesh,
        out_type=jax.ShapeDtypeStruct((B, D), jnp.float32),
        scratch_types=[
            pltpu.VMEM((b_per_w,), jnp.int32),
            pltpu.VMEM((NBUF, CH, D), jnp.float32),
        ]
        + [pltpu.SemaphoreType.DMA] * (2 * NBUF),
    )
    def gather_kernel(idx_hbm, table_hbm, out_hbm, idx_v, bufs, *sems):
        gs, ss = sems[:NBUF], sems[NBUF:]
        wid = lax.axis_index("s") * _NC + lax.axis_index("c")
        base = wid * b_per_w
        pltpu.sync_copy(idx_hbm.at[pl.ds(base, b_per_w)], idx_v)

        def gather(j, b):
            return pltpu.async_copy(
                table_hbm.at[idx_v.at[pl.ds(j * CH, CH)]], bufs.at[b], gs[b]
            )

        def store(j, b):
            return pltpu.async_copy(
                bufs.at[b], out_hbm.at[pl.ds(base + j * CH, CH)], ss[b]
            )

        gh, sh = {}, {}
        for b in range(min(NBUF, n_ch)):
            gh[b] = gather(b, b)
        for j in range(n_ch):
            b = j % NBUF
            gh[b].wait()
            sh[b] = store(j, b)
            nxt = j + NBUF
            if nxt < n_ch:
                sh[b].wait()
                gh[b] = gather(nxt, b)
        for k in range(max(0, n_ch - NBUF), n_ch):
            sh[k % NBUF].wait()

    return gather_kernel


def kernel(seq_len, table):
    V, D = table.shape
    idx = jnp.arange(V, dtype=jnp.int32) % jnp.asarray(seq_len, jnp.int32)
    return _make_gather(V, D)(idx, table)
